# baseline (speedup 1.0000x reference)
import functools

import jax
import jax.numpy as jnp
from jax import lax
from jax.experimental import pallas as pl
from jax.experimental.pallas import tpu as pltpu
from jax.experimental.pallas import tpu_sc as plsc

_L = 16


@functools.lru_cache(maxsize=None)
def _build(rows: int, n: int):
    mesh = plsc.VectorSubcoreMesh(core_axis_name="c", subcore_axis_name="s")

    @functools.partial(
        pl.kernel,
        mesh=mesh,
        out_type=jax.ShapeDtypeStruct((rows, _L), jnp.float32),
        scratch_types=[
            pltpu.VMEM((_L,), jnp.float32),
            pltpu.VMEM((_L,), jnp.int32),
            pltpu.VMEM((rows, _L), jnp.float32),
        ],
    )
    def scatter_row(upd_hbm, idx_hbm, par_hbm, out_hbm, upd_v, idx_v, out_v):
        is_worker0 = (lax.axis_index("c") == 0) & (lax.axis_index("s") == 0)

        @pl.when(is_worker0)
        def _():
            pltpu.sync_copy(upd_hbm, upd_v)
            pltpu.sync_copy(idx_hbm, idx_v)
            pltpu.sync_copy(par_hbm, out_v)
            upd = upd_v[...]
            idx = idx_v[...]
            lane = lax.iota(jnp.int32, _L)
            valid = lane < n
            for r in range(rows):
                row = out_v[r, :]
                out_v[r, :] = jnp.where((idx == r) & valid, upd, row)
            pltpu.sync_copy(out_v, out_hbm)

    return scatter_row


def kernel(update, index, params):
    rows = params.shape[0]
    n = update.shape[0]
    upd = jnp.pad(update[:, 0], (0, _L - n))
    idxb = jnp.broadcast_to(index, (_L,)).astype(jnp.int32)
    par = jnp.pad(params.reshape(rows, n), ((0, 0), (0, _L - n)))
    out = _build(rows, n)(upd, idxb, par)
    return out[:, :n].reshape(params.shape)


# speedup vs baseline: 1.0370x; 1.0370x over previous
import functools

import jax
import jax.numpy as jnp
from jax import lax
from jax.experimental import pallas as pl
from jax.experimental.pallas import tpu as pltpu
from jax.experimental.pallas import tpu_sc as plsc


@functools.lru_cache(maxsize=None)
def _build(rows: int, n: int):
    mesh = plsc.VectorSubcoreMesh(core_axis_name="c", subcore_axis_name="s")

    @functools.partial(
        pl.kernel,
        mesh=mesh,
        out_type=jax.ShapeDtypeStruct((rows, n), jnp.float32),
        scratch_types=[
            pltpu.VMEM((16,), jnp.int32),
            pltpu.VMEM((rows, n), jnp.float32),
        ],
    )
    def scatter_row(upd_hbm, idx_hbm, par_hbm, out_hbm, idx_s, buf_v):
        is_worker0 = (lax.axis_index("c") == 0) & (lax.axis_index("s") == 0)

        @pl.when(is_worker0)
        def _():
            pltpu.sync_copy(idx_hbm, idx_s.at[pl.ds(0, 1)])
            pltpu.sync_copy(par_hbm, buf_v)
            r = idx_s[...][0]
            for i in range(rows):
                @pl.when(r == i)
                def _():
                    pltpu.sync_copy(upd_hbm, buf_v.at[i])
            pltpu.sync_copy(buf_v, out_hbm)

    return scatter_row


def kernel(update, index, params):
    rows = params.shape[0]
    n = update.shape[0]
    upd = update.reshape(n)
    par = params.reshape(rows, n)
    out = _build(rows, n)(upd, index, par)
    return out.reshape(params.shape)


# speedup vs baseline: 1.1767x; 1.1347x over previous
import functools

import jax
import jax.numpy as jnp
from jax import lax
from jax.experimental import pallas as pl
from jax.experimental.pallas import tpu as pltpu
from jax.experimental.pallas import tpu_sc as plsc


@functools.lru_cache(maxsize=None)
def _build(rows: int, n: int):
    mesh = plsc.ScalarSubcoreMesh(axis_name="c", num_cores=1)

    @functools.partial(
        pl.kernel,
        mesh=mesh,
        out_type=jax.ShapeDtypeStruct((rows, n), jnp.float32),
        scratch_types=[
            pltpu.SMEM((1,), jnp.int32),
        ],
    )
    def scatter_row(upd_hbm, idx_hbm, par_hbm, out_hbm, idx_s):
        pltpu.sync_copy(idx_hbm, idx_s)
        pltpu.sync_copy(par_hbm, out_hbm)
        r = idx_s[0]
        for i in range(rows):
            @pl.when(r == i)
            def _():
                pltpu.sync_copy(upd_hbm, out_hbm.at[i])

    return scatter_row


def kernel(update, index, params):
    rows = params.shape[0]
    n = update.shape[0]
    upd = update.reshape(n)
    par = params.reshape(rows, n)
    out = _build(rows, n)(upd, index, par)
    return out.reshape(params.shape)


# speedup vs baseline: 1.2155x; 1.0329x over previous
import functools

import jax
import jax.numpy as jnp
from jax import lax
from jax.experimental import pallas as pl
from jax.experimental.pallas import tpu as pltpu
from jax.experimental.pallas import tpu_sc as plsc


@functools.lru_cache(maxsize=None)
def _build(rows: int, n: int):
    mesh = plsc.ScalarSubcoreMesh(axis_name="c", num_cores=1)

    @functools.partial(
        pl.kernel,
        mesh=mesh,
        out_type=jax.ShapeDtypeStruct((rows, n), jnp.float32),
        scratch_types=[
            pltpu.SMEM((1,), jnp.int32),
            pltpu.SemaphoreType.DMA,
        ],
    )
    def scatter_row(upd_hbm, idx_hbm, par_hbm, out_hbm, idx_s, sem):
        c1 = pltpu.async_copy(idx_hbm, idx_s, sem)
        c2 = pltpu.async_copy(par_hbm, out_hbm, sem)
        c1.wait()
        c2.wait()
        r = idx_s[0]
        for i in range(rows):
            @pl.when(r == i)
            def _():
                pltpu.sync_copy(upd_hbm, out_hbm.at[i])

    return scatter_row


def kernel(update, index, params):
    rows = params.shape[0]
    n = update.shape[0]
    upd = update.reshape(n)
    par = params.reshape(rows, n)
    out = _build(rows, n)(upd, index, par)
    return out.reshape(params.shape)
